# baseline (speedup 1.0000x reference)
import functools

import jax
import jax.numpy as jnp
from jax import lax
from jax.experimental import pallas as pl
from jax.experimental.pallas import tpu as pltpu
from jax.experimental.pallas import tpu_sc as plsc

_L = 16
_NC, _NS = 2, 16
_NW = _NC * _NS
_UNROLL = 8


def kernel(logits, alt_counts, confidence, max_logit):
    n = logits.shape[0]
    k = confidence.shape[0]
    nv = -(-(n // _L) // _NW)
    nv = -(-nv // _UNROLL) * _UNROLL
    ch = nv * _L
    kmax = k - 1

    m = max_logit.astype(jnp.float32)
    tab = jnp.zeros((_L,), jnp.float32).at[:k].set(confidence * (2.0 / m))
    pm = jnp.full((_L,), m, jnp.float32)
    p2m = jnp.full((_L,), 2.0 * m, jnp.float32)

    mesh = plsc.VectorSubcoreMesh(core_axis_name="c", subcore_axis_name="s")

    @functools.partial(
        pl.kernel,
        out_type=jax.ShapeDtypeStruct((n,), jnp.float32),
        mesh=mesh,
        compiler_params=pltpu.CompilerParams(needs_layout_passes=False),
        scratch_types=[
            pltpu.VMEM((ch,), jnp.float32),
            pltpu.VMEM((ch,), jnp.int32),
            pltpu.VMEM((ch,), jnp.float32),
            pltpu.VMEM((_L,), jnp.float32),
            pltpu.VMEM((_L,), jnp.float32),
            pltpu.VMEM((_L,), jnp.float32),
        ],
    )
    def run(logits_hbm, counts_hbm, tab_hbm, pm_hbm, p2m_hbm, out_hbm,
            lg_v, ct_v, out_v, tab_v, pm_v, p2m_v):
        wid = lax.axis_index("s") * _NC + lax.axis_index("c")
        base = jnp.minimum(wid * ch, n - ch)
        pltpu.sync_copy(tab_hbm, tab_v)
        pltpu.sync_copy(pm_hbm, pm_v)
        pltpu.sync_copy(p2m_hbm, p2m_v)
        pltpu.sync_copy(logits_hbm.at[pl.ds(base, ch)], lg_v)
        pltpu.sync_copy(counts_hbm.at[pl.ds(base, ch)], ct_v)
        pmv = pm_v[...]
        p2mv = p2m_v[...]

        @plsc.parallel_loop(0, nv, 1, unroll=_UNROLL)
        def body(i):
            x = lg_v[pl.ds(i * _L, _L)]
            ci = jnp.minimum(ct_v[pl.ds(i * _L, _L)], kmax)
            c = plsc.load_gather(tab_v, [ci])
            e = jnp.exp(x * c)
            out_v[pl.ds(i * _L, _L)] = pmv - p2mv / (e + 1.0)

        pltpu.sync_copy(out_v, out_hbm.at[pl.ds(base, ch)])

    return run(logits, alt_counts, tab, pm, p2m)


# speedup vs baseline: 1.0612x; 1.0612x over previous
import functools

import jax
import jax.numpy as jnp
from jax import lax
from jax.experimental import pallas as pl
from jax.experimental.pallas import tpu as pltpu
from jax.experimental.pallas import tpu_sc as plsc

_L = 16
_NC, _NS = 2, 16
_NW = _NC * _NS
_UNROLL = 8
_NCHUNK = 5


def kernel(logits, alt_counts, confidence, max_logit):
    n = logits.shape[0]
    k = confidence.shape[0]
    q = _UNROLL * _NCHUNK
    nv = -(-(n // _L) // _NW)
    nv = -(-nv // q) * q
    ch = nv * _L
    cnv = nv // _NCHUNK
    cch = cnv * _L
    kmax = k - 1

    m = max_logit.astype(jnp.float32)
    tab = jnp.zeros((_L,), jnp.float32).at[:k].set(confidence * (2.0 / m))
    params = jnp.concatenate(
        [tab, jnp.full((_L,), m, jnp.float32), jnp.full((_L,), 2.0 * m, jnp.float32)]
    )

    mesh = plsc.VectorSubcoreMesh(core_axis_name="c", subcore_axis_name="s")

    @functools.partial(
        pl.kernel,
        out_type=jax.ShapeDtypeStruct((n,), jnp.float32),
        mesh=mesh,
        compiler_params=pltpu.CompilerParams(needs_layout_passes=False),
        scratch_types=[
            pltpu.VMEM((cch,), jnp.float32),
            pltpu.VMEM((cch,), jnp.float32),
            pltpu.VMEM((cch,), jnp.int32),
            pltpu.VMEM((cch,), jnp.int32),
            pltpu.VMEM((cch,), jnp.float32),
            pltpu.VMEM((cch,), jnp.float32),
            pltpu.VMEM((3 * _L,), jnp.float32),
            pltpu.SemaphoreType.DMA,
            pltpu.SemaphoreType.DMA,
            pltpu.SemaphoreType.DMA,
            pltpu.SemaphoreType.DMA,
            pltpu.SemaphoreType.DMA,
            pltpu.SemaphoreType.DMA,
        ],
    )
    def run(logits_hbm, counts_hbm, params_hbm, out_hbm,
            lg0, lg1, ct0, ct1, o0, o1, par_v,
            slg0, slg1, sct0, sct1, sout0, sout1):
        lg_b = (lg0, lg1)
        ct_b = (ct0, ct1)
        out_b = (o0, o1)
        slg = (slg0, slg1)
        sct = (sct0, sct1)
        sout = (sout0, sout1)
        wid = lax.axis_index("s") * _NC + lax.axis_index("c")
        base = jnp.minimum(wid * ch, n - ch)
        pltpu.sync_copy(params_hbm, par_v)
        tabr = par_v.at[pl.ds(0, _L)]
        pmv = par_v[pl.ds(_L, _L)]
        p2mv = par_v[pl.ds(2 * _L, _L)]

        def start_in(j):
            b = j % 2
            hl = pltpu.async_copy(
                logits_hbm.at[pl.ds(base + j * cch, cch)], lg_b[b], slg[b])
            hc = pltpu.async_copy(
                counts_hbm.at[pl.ds(base + j * cch, cch)], ct_b[b], sct[b])
            return hl, hc

        hin = [None] * _NCHUNK
        hout = [None] * _NCHUNK
        hin[0] = start_in(0)
        for j in range(_NCHUNK):
            if j + 1 < _NCHUNK:
                hin[j + 1] = start_in(j + 1)
            hin[j][0].wait()
            hin[j][1].wait()
            if j >= 2:
                hout[j - 2].wait()
            b = j % 2
            lgb, ctb, outb = lg_b[b], ct_b[b], out_b[b]

            @plsc.parallel_loop(0, cnv, 1, unroll=_UNROLL)
            def body(i):
                x = lgb[pl.ds(i * _L, _L)]
                ci = jnp.minimum(ctb[pl.ds(i * _L, _L)], kmax)
                c = plsc.load_gather(tabr, [ci])
                e = jnp.exp(x * c)
                outb[pl.ds(i * _L, _L)] = pmv - p2mv / (e + 1.0)

            hout[j] = pltpu.async_copy(
                outb, out_hbm.at[pl.ds(base + j * cch, cch)], sout[b])
        hout[_NCHUNK - 2].wait()
        hout[_NCHUNK - 1].wait()

    return run(logits, alt_counts, params)
